# baseline (speedup 1.0000x reference)
import functools

import numpy as np
import jax
import jax.numpy as jnp
from jax import lax
from jax.experimental import pallas as pl
from jax.experimental.pallas import tpu as pltpu

S, D = 2048, 1024
H, DH = 16, 64
E, K, F = 8, 2, 1408
FS = 2816
EPS = 1e-6
THETA = 10000.0
HD = H * DH

BS = 256
GT = 128
NT = 40
P_PAD = 5376
TRASH_BLK = P_PAD // GT - 1

_INTERPRET = False


def _rope_tables():
    pos = np.arange(S, dtype=np.float64)
    inv = 1.0 / (THETA ** (np.arange(0, DH, 2, dtype=np.float64) / DH))
    ang = pos[:, None] * inv[None, :]
    cos = np.concatenate([np.cos(ang), np.cos(ang)], axis=-1)
    sin = np.concatenate([np.sin(ang), np.sin(ang)], axis=-1)
    cos_full = np.tile(cos, (1, H)).astype(np.float32)
    sin_full = np.tile(sin, (1, H)).astype(np.float32)
    prot = np.zeros((HD, HD), dtype=np.float32)
    for h in range(H):
        for c in range(DH):
            j = h * DH + c
            if c < DH // 2:
                prot[h * DH + c + DH // 2, j] = -1.0
            else:
                prot[h * DH + c - DH // 2, j] = 1.0
    return jnp.asarray(cos_full), jnp.asarray(sin_full), jnp.asarray(prot)


def _pre_attn_body(x_ref, w_ref, wq_ref, bq_ref, wk_ref, bk_ref, wv_ref,
                   bv_ref, cos_ref, sin_ref, prot_ref, q_ref, k_ref, v_ref):
    x = x_ref[...]
    ms = jnp.mean(x * x, axis=-1, keepdims=True)
    xn = x * lax.rsqrt(ms + EPS) * w_ref[...]
    q = jnp.dot(xn, wq_ref[...], preferred_element_type=jnp.float32) + bq_ref[...]
    k = jnp.dot(xn, wk_ref[...], preferred_element_type=jnp.float32) + bk_ref[...]
    v = jnp.dot(xn, wv_ref[...], preferred_element_type=jnp.float32) + bv_ref[...]
    cos = cos_ref[...]
    sin = sin_ref[...]
    prot = prot_ref[...]
    q_ref[...] = q * cos + jnp.dot(q, prot, preferred_element_type=jnp.float32) * sin
    k_ref[...] = k * cos + jnp.dot(k, prot, preferred_element_type=jnp.float32) * sin
    v_ref[...] = v


def _pre_attn(x, ln1_w, wq, bq, wk, bk, wv, bv, cos_f, sin_f, prot):
    row = pl.BlockSpec((BS, D), lambda i: (i, 0))
    full = pl.BlockSpec((D, HD), lambda i: (0, 0))
    vec = pl.BlockSpec((1, HD), lambda i: (0, 0))
    vecd = pl.BlockSpec((1, D), lambda i: (0, 0))
    out = jax.ShapeDtypeStruct((S, HD), jnp.float32)
    return pl.pallas_call(
        _pre_attn_body,
        grid=(S // BS,),
        in_specs=[row, vecd, full, vec, full, vec, full, vec,
                  pl.BlockSpec((BS, HD), lambda i: (i, 0)),
                  pl.BlockSpec((BS, HD), lambda i: (i, 0)),
                  pl.BlockSpec((HD, HD), lambda i: (0, 0))],
        out_specs=[pl.BlockSpec((BS, HD), lambda i: (i, 0))] * 3,
        out_shape=[out, out, out],
        interpret=_INTERPRET,
    )(x, ln1_w.reshape(1, D), wq, bq.reshape(1, HD), wk, bk.reshape(1, HD),
      wv, bv.reshape(1, HD), cos_f, sin_f, prot)


def _flash_body(q_ref, k_ref, v_ref, o_ref):
    i = pl.program_id(1)
    q = q_ref[0]
    scale = DH ** -0.5

    def step(j, carry):
        m, l, acc = carry
        kj = k_ref[0, pl.ds(j * BS, BS), :]
        vj = v_ref[0, pl.ds(j * BS, BS), :]
        s = lax.dot_general(q, kj, (((1,), (1,)), ((), ())),
                            preferred_element_type=jnp.float32) * scale
        rows = i * BS + lax.broadcasted_iota(jnp.int32, (BS, BS), 0)
        cols = j * BS + lax.broadcasted_iota(jnp.int32, (BS, BS), 1)
        s = jnp.where(cols <= rows, s, -1e9)
        m_new = jnp.maximum(m, jnp.max(s, axis=1, keepdims=True))
        alpha = jnp.exp(m - m_new)
        p = jnp.exp(s - m_new)
        l_new = l * alpha + jnp.sum(p, axis=1, keepdims=True)
        acc_new = acc * alpha + jnp.dot(p, vj, preferred_element_type=jnp.float32)
        return m_new, l_new, acc_new

    m0 = jnp.full((BS, 1), -1e30, jnp.float32)
    l0 = jnp.zeros((BS, 1), jnp.float32)
    a0 = jnp.zeros((BS, DH), jnp.float32)
    m, l, acc = lax.fori_loop(0, i + 1, step, (m0, l0, a0))
    o_ref[0] = acc / l


def _flash_attn(qh, kh, vh):
    qspec = pl.BlockSpec((1, BS, DH), lambda h, i: (h, i, 0))
    kspec = pl.BlockSpec((1, S, DH), lambda h, i: (h, 0, 0))
    return pl.pallas_call(
        _flash_body,
        grid=(H, S // BS),
        in_specs=[qspec, kspec, kspec],
        out_specs=qspec,
        out_shape=jax.ShapeDtypeStruct((H, S, DH), jnp.float32),
        interpret=_INTERPRET,
    )(qh, kh, vh)


def _post_attn_body(o_ref, wo_ref, res_ref, ln2_ref, wr_ref, tril_ref,
                    h_ref, h2_ref, e1_ref, e2_ref, w1_ref, w2_ref,
                    r1_ref, r2_ref, tot_ref, base_ref):
    pid = pl.program_id(0)

    @pl.when(pid == 0)
    def _():
        base_ref[...] = jnp.zeros_like(base_ref)

    h = res_ref[...] + jnp.dot(o_ref[...], wo_ref[...],
                               preferred_element_type=jnp.float32)
    h_ref[...] = h
    ms = jnp.mean(h * h, axis=-1, keepdims=True)
    h2 = h * lax.rsqrt(ms + EPS) * ln2_ref[...]
    h2_ref[...] = h2
    logits = jnp.dot(h2, wr_ref[...], preferred_element_type=jnp.float32)
    mx = jnp.max(logits, axis=-1, keepdims=True)
    ex = jnp.exp(logits - mx)
    p = ex / jnp.sum(ex, axis=-1, keepdims=True)

    lane = lax.broadcasted_iota(jnp.int32, (BS, E), 1)
    m1 = jnp.max(p, axis=1, keepdims=True)
    i1 = jnp.min(jnp.where(p == m1, lane, E), axis=1, keepdims=True)
    oh1 = lane == i1
    p2 = jnp.where(oh1, -jnp.inf, p)
    m2 = jnp.max(p2, axis=1, keepdims=True)
    i2 = jnp.min(jnp.where(p2 == m2, lane, E), axis=1, keepdims=True)
    oh2 = lane == i2

    onehot2 = jnp.where(oh1 | oh2, 1.0, 0.0)
    cnt_in = jnp.dot(tril_ref[...], onehot2,
                     preferred_element_type=jnp.float32)
    cnt = base_ref[...] + cnt_in
    r1 = jnp.sum(jnp.where(oh1, cnt, 0.0), axis=1, keepdims=True)
    r2 = jnp.sum(jnp.where(oh2, cnt, 0.0), axis=1, keepdims=True)

    e1_ref[...] = i1
    e2_ref[...] = i2
    w1_ref[...] = m1
    w2_ref[...] = m2
    r1_ref[...] = r1.astype(jnp.int32)
    r2_ref[...] = r2.astype(jnp.int32)
    new_base = base_ref[...] + jnp.sum(onehot2, axis=0, keepdims=True)
    base_ref[...] = new_base
    tot_ref[...] = new_base.astype(jnp.int32)


def _post_attn(o_flat, wo, resid, ln2_w, w_router, tril):
    row = pl.BlockSpec((BS, D), lambda i: (i, 0))
    col1 = pl.BlockSpec((BS, 1), lambda i: (i, 0))
    outs = [
        jax.ShapeDtypeStruct((S, D), jnp.float32),
        jax.ShapeDtypeStruct((S, D), jnp.float32),
        jax.ShapeDtypeStruct((S, 1), jnp.int32),
        jax.ShapeDtypeStruct((S, 1), jnp.int32),
        jax.ShapeDtypeStruct((S, 1), jnp.float32),
        jax.ShapeDtypeStruct((S, 1), jnp.float32),
        jax.ShapeDtypeStruct((S, 1), jnp.int32),
        jax.ShapeDtypeStruct((S, 1), jnp.int32),
        jax.ShapeDtypeStruct((1, E), jnp.int32),
    ]
    return pl.pallas_call(
        _post_attn_body,
        grid=(S // BS,),
        in_specs=[row,
                  pl.BlockSpec((HD, D), lambda i: (0, 0)),
                  row,
                  pl.BlockSpec((1, D), lambda i: (0, 0)),
                  pl.BlockSpec((D, E), lambda i: (0, 0)),
                  pl.BlockSpec((BS, BS), lambda i: (0, 0))],
        out_specs=[row, row, col1, col1, col1, col1, col1, col1,
                   pl.BlockSpec((1, E), lambda i: (0, 0))],
        out_shape=outs,
        scratch_shapes=[pltpu.VMEM((1, E), jnp.float32)],
        interpret=_INTERPRET,
    )(o_flat, wo, resid, ln2_w.reshape(1, D), w_router, tril)


def _ffn_body(emap_ref, bmap_ref, act_ref, x_ref, sc_ref, wg_ref, wu_ref,
              wd_ref, y_ref):
    i = pl.program_id(0)

    @pl.when(act_ref[i] != 0)
    def _():
        x = x_ref[...]
        g = jnp.dot(x, wg_ref[0], preferred_element_type=jnp.float32)
        u = jnp.dot(x, wu_ref[0], preferred_element_type=jnp.float32)
        mid = g * lax.logistic(g) * u
        y = jnp.dot(mid, wd_ref[0], preferred_element_type=jnp.float32)
        y_ref[...] = y * sc_ref[...]


def _grouped_ffn(emap, bmap, act, x_sorted, scale2d, w_gate, w_up, w_down):
    grid_spec = pltpu.PrefetchScalarGridSpec(
        num_scalar_prefetch=3,
        grid=(NT,),
        in_specs=[
            pl.BlockSpec((GT, D), lambda i, em, bm, ac: (bm[i], 0)),
            pl.BlockSpec((GT, 1), lambda i, em, bm, ac: (bm[i], 0)),
            pl.BlockSpec((1, D, F), lambda i, em, bm, ac: (em[i], 0, 0)),
            pl.BlockSpec((1, D, F), lambda i, em, bm, ac: (em[i], 0, 0)),
            pl.BlockSpec((1, F, D), lambda i, em, bm, ac: (em[i], 0, 0)),
        ],
        out_specs=pl.BlockSpec((GT, D), lambda i, em, bm, ac: (bm[i], 0)),
    )
    return pl.pallas_call(
        _ffn_body,
        grid_spec=grid_spec,
        out_shape=jax.ShapeDtypeStruct((P_PAD, D), jnp.float32),
        compiler_params=pltpu.CompilerParams(vmem_limit_bytes=100 * 1024 * 1024),
        interpret=_INTERPRET,
    )(emap, bmap, act, x_sorted, scale2d, w_gate, w_up, w_down)


def _shared_body(x_ref, h_ref, y1_ref, y2_ref, wsg_ref, wsu_ref, wsd_ref,
                 wsh_ref, out_ref):
    x = x_ref[...]
    g = jnp.dot(x, wsg_ref[...], preferred_element_type=jnp.float32)
    u = jnp.dot(x, wsu_ref[...], preferred_element_type=jnp.float32)
    mid = g * lax.logistic(g) * u
    sh = jnp.dot(mid, wsd_ref[...], preferred_element_type=jnp.float32)
    gl = jnp.sum(x * wsh_ref[...], axis=1, keepdims=True)
    gate = lax.logistic(gl)
    out_ref[...] = h_ref[...] + y1_ref[...] + y2_ref[...] + gate * sh


def _shared_final(h2, h, ypairs, ws_gate, ws_up, ws_down, w_shgate):
    row = pl.BlockSpec((BS, D), lambda i: (i, 0))
    return pl.pallas_call(
        _shared_body,
        grid=(S // BS,),
        in_specs=[row, row,
                  pl.BlockSpec((BS, D), lambda i: (i, 0)),
                  pl.BlockSpec((BS, D), lambda i: (i + S // BS, 0)),
                  pl.BlockSpec((D, FS), lambda i: (0, 0)),
                  pl.BlockSpec((D, FS), lambda i: (0, 0)),
                  pl.BlockSpec((FS, D), lambda i: (0, 0)),
                  pl.BlockSpec((1, D), lambda i: (0, 0))],
        out_specs=row,
        out_shape=jax.ShapeDtypeStruct((S, D), jnp.float32),
        compiler_params=pltpu.CompilerParams(vmem_limit_bytes=100 * 1024 * 1024),
        interpret=_INTERPRET,
    )(h2, h, ypairs, ypairs, ws_gate, ws_up, ws_down, w_shgate.reshape(1, D))


def _route_scatter(e1, e2, r1, r2, w1, w2, a16):
    t = jnp.arange(S, dtype=jnp.int32)
    pos1 = r1 + a16[e1]
    pos2 = r2 + a16[e2]
    tok_src = jnp.zeros((P_PAD,), jnp.int32).at[pos1].set(t).at[pos2].set(t)
    scale = jnp.zeros((P_PAD,), jnp.float32).at[pos1].set(w1).at[pos2].set(w2)
    return tok_src, scale, pos1, pos2


def _row_gather(table, idx):
    return table[idx]


def kernel(hidden_states, ln1_w, wq, bq, wk, bk, wv, bv, wo, ln2_w,
           w_router, w_gate, w_up, w_down, ws_gate, ws_up, ws_down, w_shgate):
    x = hidden_states.reshape(S, D)
    cos_f, sin_f, prot = _rope_tables()
    tril = jnp.asarray(
        np.tril(np.ones((BS, BS), np.float32), k=-1))

    q, k, v = _pre_attn(x, ln1_w, wq, bq, wk, bk, wv, bv, cos_f, sin_f, prot)
    qh = q.reshape(S, H, DH).transpose(1, 0, 2)
    kh = k.reshape(S, H, DH).transpose(1, 0, 2)
    vh = v.reshape(S, H, DH).transpose(1, 0, 2)
    o = _flash_attn(qh, kh, vh)
    o_flat = o.transpose(1, 0, 2).reshape(S, HD)

    h, h2, e1, e2, w1, w2, r1, r2, totals = _post_attn(
        o_flat, wo, x, ln2_w, w_router, tril)

    n = totals.reshape(E)
    blocks = (n + GT - 1) // GT
    cumb = jnp.cumsum(blocks)
    starts = cumb - blocks
    total_blocks = cumb[-1]
    a16 = jnp.zeros((16,), jnp.int32).at[:E].set(starts * GT)
    steps = jnp.arange(NT, dtype=jnp.int32)
    act = (steps < total_blocks).astype(jnp.int32)
    emap = jnp.sum((steps[:, None] >= cumb[None, :]).astype(jnp.int32), axis=1)
    emap = jnp.where(act == 1, emap, 0).astype(jnp.int32)
    bmap = jnp.where(act == 1, steps, TRASH_BLK).astype(jnp.int32)

    tok_src, scale, pos1, pos2 = _route_scatter(
        e1.reshape(S), e2.reshape(S), r1.reshape(S), r2.reshape(S),
        w1.reshape(S), w2.reshape(S), a16)

    x_sorted = _row_gather(h2, tok_src)
    y = _grouped_ffn(emap, bmap, act, x_sorted, scale.reshape(P_PAD, 1),
                     w_gate, w_up, w_down)
    ypairs = _row_gather(y, jnp.concatenate([pos1, pos2]))

    out = _shared_final(h2, h, ypairs, ws_gate, ws_up, ws_down, w_shgate)
    return out.reshape(1, S, D)
